# baseline (speedup 1.0000x reference)
import jax
import jax.numpy as jnp
from jax import lax
from jax.experimental import pallas as pl
from jax.experimental.pallas import tpu as pltpu
from jax.experimental.pallas import tpu_sc as plsc

_B, _F, _P = 128, 4, 32768
_K = 1024
_NW = 32
_RPW = _B // _NW
_NVROW = _P // 16
_U = 8

_mesh = plsc.VectorSubcoreMesh(core_axis_name="c", subcore_axis_name="s")


def _body(xf_hbm, out_hbm, v_ref, ca_ref, cb_ref, hist_ref, base_ref,
          ka_ref, ia_ref, kb_ref, ib_ref, gi_ref, gd_ref, sem):
    wid = lax.axis_index("s") * 2 + lax.axis_index("c")
    iota = lax.iota(jnp.int32, 16)
    ones = jnp.ones(16, jnp.int32)
    zeros = jnp.zeros(16, jnp.int32)

    def keys_of(v):
        b = plsc.bitcast(v, jnp.int32)
        flip = (b >> 31) | jnp.int32(-2147483648)
        return plsc.bitcast(b ^ flip, jnp.uint32)

    def zero_hist():
        for j in range(16):
            hist_ref[pl.ds(j * 16, 16)] = zeros

    def find_bin(p):
        z = jnp.int32(0)
        found, bbin, below, cnt_at, run = z, z, z, z, z
        for j in range(16):
            h = hist_ref[pl.ds(j * 16, 16)]
            cum = plsc.cumsum(h) + run
            ffs = jnp.max(plsc.all_reduce_ffs(cum > p))
            newly = (found == 0) & (ffs < 16)
            lane_cnt = jnp.sum(jnp.where(iota == ffs, h, 0))
            lane_cum = jnp.sum(jnp.where(iota == ffs, cum, 0))
            bbin = jnp.where(newly, j * 16 + ffs, bbin)
            below = jnp.where(newly, lane_cum - lane_cnt, below)
            cnt_at = jnp.where(newly, lane_cnt, cnt_at)
            run = run + jnp.sum(h)
            found = found | newly.astype(jnp.int32)
        return bbin, below, cnt_at

    def per_row(rr, row_carry):
        row = wid * _RPW + rr
        row_base = row * (_F * _P)
        pltpu.sync_copy(xf_hbm.at[pl.ds(row_base, _P)], v_ref)

        zero_hist()

        def h0(j, c):
            for u in range(_U):
                k = keys_of(v_ref[pl.ds((j * _U + u) * 16, 16)])
                b = (k >> 24).astype(jnp.int32)
                cnt, last = plsc.scan_count(b)
                plsc.addupdate_scatter(hist_ref, [b], cnt, mask=last)
            return c

        lax.fori_loop(0, _NVROW // _U, h0, jnp.int32(0))

        above = jnp.int32(0)
        cn = jnp.int32(_P)
        bbin, below, cnt_at = find_bin(cn - (_K - above))
        above = above + (cn - below - cnt_at)
        cn = cnt_at
        t_key = bbin.astype(jnp.uint32) << 24

        def c0(j, w):
            for u in range(_U):
                k = keys_of(v_ref[pl.ds((j * _U + u) * 16, 16)])
                m = (k >> 24).astype(jnp.int32) == bbin
                cs = plsc.cumsum(m.astype(jnp.int32))
                plsc.store_scatter(ca_ref, [w + cs - 1],
                                   plsc.bitcast(k, jnp.int32), mask=m)
                w = w + jnp.max(cs)
            return w

        lax.fori_loop(0, _NVROW // _U, c0, jnp.int32(0))

        for lvl, shift in enumerate((16, 8, 0)):
            src = (ca_ref, cb_ref)[lvl % 2]
            dst = (cb_ref, ca_ref)[lvl % 2]
            nv = (cn + 16 * _U - 1) // (16 * _U)
            zero_hist()

            def hl(j, c, src=src, shift=shift, cn=cn):
                for u in range(_U):
                    base = (j * _U + u) * 16
                    k = plsc.bitcast(src[pl.ds(base, 16)], jnp.uint32)
                    valid = (base + iota) < cn
                    b = ((k >> shift) & 0xFF).astype(jnp.int32)
                    cnt, last = plsc.scan_count(b, mask=valid)
                    plsc.addupdate_scatter(hist_ref, [b], cnt, mask=last)
                return c

            lax.fori_loop(0, nv, hl, jnp.int32(0))
            bbin, below, cnt_at = find_bin(cn - (_K - above))

            if shift != 0:
                def cl(j, w, src=src, dst=dst, shift=shift, cn=cn,
                       bbin=bbin):
                    for u in range(_U):
                        base = (j * _U + u) * 16
                        ki = src[pl.ds(base, 16)]
                        k = plsc.bitcast(ki, jnp.uint32)
                        valid = (base + iota) < cn
                        m = valid & (((k >> shift) & 0xFF).astype(jnp.int32)
                                     == bbin)
                        cs = plsc.cumsum(m.astype(jnp.int32))
                        plsc.store_scatter(dst, [w + cs - 1], ki, mask=m)
                        w = w + jnp.max(cs)
                    return w

                lax.fori_loop(0, nv, cl, jnp.int32(0))

            above = above + (cn - below - cnt_at)
            cn = cnt_at
            t_key = t_key | (bbin.astype(jnp.uint32) << shift)

        t_skip = cn - (_K - above)

        def sel(j, c):
            ts, ns = c
            for u in range(_U):
                base = (j * _U + u) * 16
                k = keys_of(v_ref[pl.ds(base, 16)])
                m_gt = k > t_key
                m_eq = k == t_key
                ce = plsc.cumsum(m_eq.astype(jnp.int32))
                keep = m_gt | (m_eq & ((ts + ce - 1) >= t_skip))
                ck = plsc.cumsum(keep.astype(jnp.int32))
                pos = ns + ck - 1
                plsc.store_scatter(ka_ref, [pos],
                                   plsc.bitcast(k, jnp.int32), mask=keep)
                plsc.store_scatter(ia_ref, [pos], base + iota, mask=keep)
                ts = ts + jnp.max(ce)
                ns = ns + jnp.max(ck)
            return ts, ns

        lax.fori_loop(0, _NVROW // _U, sel, (jnp.int32(0), jnp.int32(0)))

        bufs = ((ka_ref, ia_ref), (kb_ref, ib_ref))
        for p in range(4):
            src_k, src_i = bufs[p % 2]
            dst_k, dst_i = bufs[(p + 1) % 2]
            shift = 8 * p
            zero_hist()

            def hp(j, c, src_k=src_k, shift=shift):
                for u in range(_U):
                    k = plsc.bitcast(
                        src_k[pl.ds((j * _U + u) * 16, 16)], jnp.uint32)
                    b = ((k >> shift) & 0xFF).astype(jnp.int32)
                    cnt, last = plsc.scan_count(b)
                    plsc.addupdate_scatter(hist_ref, [b], cnt, mask=last)
                return c

            lax.fori_loop(0, _K // 16 // _U, hp, jnp.int32(0))

            run = jnp.int32(0)
            for j in range(16):
                h = hist_ref[pl.ds(j * 16, 16)]
                cum = plsc.cumsum(h) + run
                base_ref[pl.ds(j * 16, 16)] = cum - h
                run = jnp.max(cum)

            def sc(j, c, src_k=src_k, src_i=src_i, dst_k=dst_k,
                   dst_i=dst_i, shift=shift):
                for u in range(_U):
                    base = (j * _U + u) * 16
                    ki = src_k[pl.ds(base, 16)]
                    ix = src_i[pl.ds(base, 16)]
                    b = ((plsc.bitcast(ki, jnp.uint32) >> shift)
                         & 0xFF).astype(jnp.int32)
                    cnt, last = plsc.scan_count(b)
                    pos = plsc.load_gather(base_ref, [b]) + cnt - 1
                    plsc.store_scatter(dst_k, [pos], ki)
                    plsc.store_scatter(dst_i, [pos], ix)
                    plsc.addupdate_scatter(base_ref, [b], cnt, mask=last)
                return c

            lax.fori_loop(0, _K // 16 // _U, sc, jnp.int32(0))

        for f in range(_F):
            off = row_base + f * _P

            def gx(j, c, off=off):
                for u in range(_U):
                    base = (j * _U + u) * 16
                    gi_ref[pl.ds(base, 16)] = ia_ref[pl.ds(base, 16)] + off
                return c

            lax.fori_loop(0, _K // 16 // _U, gx, jnp.int32(0))
            copies = [
                pltpu.async_copy(
                    xf_hbm.at[gi_ref.at[pl.ds(c * 128, 128)]],
                    gd_ref.at[pl.ds(c * 128, 128)], sem)
                for c in range(_K // 128)
            ]
            for cp in copies:
                cp.wait()
            pltpu.sync_copy(gd_ref,
                            out_hbm.at[pl.ds((row * _F + f) * _K, _K)])
        return row_carry

    lax.fori_loop(0, _RPW, per_row, jnp.int32(0))


_kernel_call = pl.kernel(
    _body,
    out_type=jax.ShapeDtypeStruct((_B * _F * _K,), jnp.float32),
    mesh=_mesh,
    compiler_params=pltpu.CompilerParams(needs_layout_passes=False),
    scratch_types=[
        pltpu.VMEM((_P,), jnp.float32),
        pltpu.VMEM((_P,), jnp.int32),
        pltpu.VMEM((_P,), jnp.int32),
        pltpu.VMEM((256,), jnp.int32),
        pltpu.VMEM((256,), jnp.int32),
        pltpu.VMEM((_K,), jnp.int32),
        pltpu.VMEM((_K,), jnp.int32),
        pltpu.VMEM((_K,), jnp.int32),
        pltpu.VMEM((_K,), jnp.int32),
        pltpu.VMEM((_K,), jnp.int32),
        pltpu.VMEM((_K,), jnp.float32),
        pltpu.SemaphoreType.DMA,
    ],
)


@jax.jit
def kernel(x):
    out = _kernel_call(x.reshape(_B * _F * _P))
    return out.reshape(_B, _F, _K)


# speedup vs baseline: 1.1531x; 1.1531x over previous
import jax
import jax.numpy as jnp
from jax import lax
from jax.experimental import pallas as pl
from jax.experimental.pallas import tpu as pltpu
from jax.experimental.pallas import tpu_sc as plsc

_B, _F, _P = 128, 4, 32768
_K = 1024
_NW = 32
_RPW = _B // _NW
_NVROW = _P // 16
_U = 8

_mesh = plsc.VectorSubcoreMesh(core_axis_name="c", subcore_axis_name="s")


def _body(xf_hbm, out_hbm, v_ref, ca_ref, cb_ref, hist_ref, base_ref,
          ka_ref, ia_ref, kb_ref, ib_ref, gi_ref, gd_ref, sem):
    wid = lax.axis_index("s") * 2 + lax.axis_index("c")
    iota = lax.iota(jnp.int32, 16)
    ones = jnp.ones(16, jnp.int32)
    zeros = jnp.zeros(16, jnp.int32)

    def keys_of(v):
        b = plsc.bitcast(v, jnp.int32)
        flip = (b >> 31) | jnp.int32(-2147483648)
        return plsc.bitcast(b ^ flip, jnp.uint32)

    def zero_hist():
        for j in range(16):
            hist_ref[pl.ds(j * 16, 16)] = zeros

    def find_bin(p):
        z = jnp.int32(0)
        found, bbin, below, cnt_at, run = z, z, z, z, z
        for j in range(16):
            h = hist_ref[pl.ds(j * 16, 16)]
            cum = plsc.cumsum(h) + run
            ffs = jnp.max(plsc.all_reduce_ffs(cum > p))
            newly = (found == 0) & (ffs < 16)
            lane_cnt = jnp.sum(jnp.where(iota == ffs, h, 0))
            lane_cum = jnp.sum(jnp.where(iota == ffs, cum, 0))
            bbin = jnp.where(newly, j * 16 + ffs, bbin)
            below = jnp.where(newly, lane_cum - lane_cnt, below)
            cnt_at = jnp.where(newly, lane_cnt, cnt_at)
            run = run + jnp.sum(h)
            found = found | newly.astype(jnp.int32)
        return bbin, below, cnt_at

    def per_row(rr, row_carry):
        row = wid * _RPW + rr
        row_base = row * (_F * _P)
        pltpu.sync_copy(xf_hbm.at[pl.ds(row_base, _P)], v_ref)

        zero_hist()

        def h0(j, c):
            for u in range(_U):
                k = keys_of(v_ref[pl.ds((j * _U + u) * 16, 16)])
                plsc.addupdate_scatter(
                    hist_ref, [(k >> 24).astype(jnp.int32)], ones)
            return c

        lax.fori_loop(0, _NVROW // _U, h0, jnp.int32(0))

        above = jnp.int32(0)
        cn = jnp.int32(_P)
        bbin, below, cnt_at = find_bin(cn - (_K - above))
        above = above + (cn - below - cnt_at)
        cn = cnt_at
        t_key = bbin.astype(jnp.uint32) << 24

        def c0(j, w):
            for u in range(_U):
                k = keys_of(v_ref[pl.ds((j * _U + u) * 16, 16)])
                m = (k >> 24).astype(jnp.int32) == bbin
                cs = plsc.cumsum(m.astype(jnp.int32))
                plsc.store_scatter(ca_ref, [w + cs - 1],
                                   plsc.bitcast(k, jnp.int32), mask=m)
                w = w + jnp.max(cs)
            return w

        lax.fori_loop(0, _NVROW // _U, c0, jnp.int32(0))

        for lvl, shift in enumerate((16, 8, 0)):
            src = (ca_ref, cb_ref)[lvl % 2]
            dst = (cb_ref, ca_ref)[lvl % 2]
            nv = (cn + 16 * _U - 1) // (16 * _U)
            zero_hist()

            def hl(j, c, src=src, shift=shift, cn=cn):
                for u in range(_U):
                    base = (j * _U + u) * 16
                    k = plsc.bitcast(src[pl.ds(base, 16)], jnp.uint32)
                    valid = (base + iota) < cn
                    b = ((k >> shift) & 0xFF).astype(jnp.int32)
                    plsc.addupdate_scatter(hist_ref, [b], ones, mask=valid)
                return c

            lax.fori_loop(0, nv, hl, jnp.int32(0))
            bbin, below, cnt_at = find_bin(cn - (_K - above))

            if shift != 0:
                def cl(j, w, src=src, dst=dst, shift=shift, cn=cn,
                       bbin=bbin):
                    for u in range(_U):
                        base = (j * _U + u) * 16
                        ki = src[pl.ds(base, 16)]
                        k = plsc.bitcast(ki, jnp.uint32)
                        valid = (base + iota) < cn
                        m = valid & (((k >> shift) & 0xFF).astype(jnp.int32)
                                     == bbin)
                        cs = plsc.cumsum(m.astype(jnp.int32))
                        plsc.store_scatter(dst, [w + cs - 1], ki, mask=m)
                        w = w + jnp.max(cs)
                    return w

                lax.fori_loop(0, nv, cl, jnp.int32(0))

            above = above + (cn - below - cnt_at)
            cn = cnt_at
            t_key = t_key | (bbin.astype(jnp.uint32) << shift)

        t_skip = cn - (_K - above)

        def sel(j, c):
            ts, ns = c
            for u in range(_U):
                base = (j * _U + u) * 16
                k = keys_of(v_ref[pl.ds(base, 16)])
                m_gt = k > t_key
                m_eq = k == t_key
                ce = plsc.cumsum(m_eq.astype(jnp.int32))
                keep = m_gt | (m_eq & ((ts + ce - 1) >= t_skip))
                ck = plsc.cumsum(keep.astype(jnp.int32))
                pos = ns + ck - 1
                plsc.store_scatter(ka_ref, [pos],
                                   plsc.bitcast(k, jnp.int32), mask=keep)
                plsc.store_scatter(ia_ref, [pos], base + iota, mask=keep)
                ts = ts + jnp.max(ce)
                ns = ns + jnp.max(ck)
            return ts, ns

        lax.fori_loop(0, _NVROW // _U, sel, (jnp.int32(0), jnp.int32(0)))

        bufs = ((ka_ref, ia_ref), (kb_ref, ib_ref))
        for p in range(4):
            src_k, src_i = bufs[p % 2]
            dst_k, dst_i = bufs[(p + 1) % 2]
            shift = 8 * p
            zero_hist()

            def hp(j, c, src_k=src_k, shift=shift):
                for u in range(_U):
                    k = plsc.bitcast(
                        src_k[pl.ds((j * _U + u) * 16, 16)], jnp.uint32)
                    b = ((k >> shift) & 0xFF).astype(jnp.int32)
                    plsc.addupdate_scatter(hist_ref, [b], ones)
                return c

            lax.fori_loop(0, _K // 16 // _U, hp, jnp.int32(0))

            run = jnp.int32(0)
            for j in range(16):
                h = hist_ref[pl.ds(j * 16, 16)]
                cum = plsc.cumsum(h) + run
                base_ref[pl.ds(j * 16, 16)] = cum - h
                run = jnp.max(cum)

            def sc(j, c, src_k=src_k, src_i=src_i, dst_k=dst_k,
                   dst_i=dst_i, shift=shift):
                for u in range(_U):
                    base = (j * _U + u) * 16
                    ki = src_k[pl.ds(base, 16)]
                    ix = src_i[pl.ds(base, 16)]
                    b = ((plsc.bitcast(ki, jnp.uint32) >> shift)
                         & 0xFF).astype(jnp.int32)
                    cnt, _ = plsc.scan_count(b)
                    pos = plsc.load_gather(base_ref, [b]) + cnt - 1
                    plsc.store_scatter(dst_k, [pos], ki)
                    plsc.store_scatter(dst_i, [pos], ix)
                    plsc.addupdate_scatter(base_ref, [b], ones)
                return c

            lax.fori_loop(0, _K // 16 // _U, sc, jnp.int32(0))

        def gx(j, c):
            for u in range(_U):
                base = (j * _U + u) * 16
                ix = ia_ref[pl.ds(base, 16)]
                for f in range(_F):
                    gi_ref[pl.ds(f * _K + base, 16)] = (
                        ix + (row_base + f * _P))
            return c

        lax.fori_loop(0, _K // 16 // _U, gx, jnp.int32(0))
        copies = [
            pltpu.async_copy(
                xf_hbm.at[gi_ref.at[pl.ds(c * 128, 128)]],
                gd_ref.at[pl.ds(c * 128, 128)], sem)
            for c in range(_F * _K // 128)
        ]
        for cp in copies:
            cp.wait()
        pltpu.sync_copy(gd_ref,
                        out_hbm.at[pl.ds(row * _F * _K, _F * _K)])
        return row_carry

    lax.fori_loop(0, _RPW, per_row, jnp.int32(0))


_kernel_call = pl.kernel(
    _body,
    out_type=jax.ShapeDtypeStruct((_B * _F * _K,), jnp.float32),
    mesh=_mesh,
    compiler_params=pltpu.CompilerParams(needs_layout_passes=False),
    scratch_types=[
        pltpu.VMEM((_P,), jnp.float32),
        pltpu.VMEM((_P,), jnp.int32),
        pltpu.VMEM((_P,), jnp.int32),
        pltpu.VMEM((256,), jnp.int32),
        pltpu.VMEM((256,), jnp.int32),
        pltpu.VMEM((_K,), jnp.int32),
        pltpu.VMEM((_K,), jnp.int32),
        pltpu.VMEM((_K,), jnp.int32),
        pltpu.VMEM((_K,), jnp.int32),
        pltpu.VMEM((_F * _K,), jnp.int32),
        pltpu.VMEM((_F * _K,), jnp.float32),
        pltpu.SemaphoreType.DMA,
    ],
)


@jax.jit
def kernel(x):
    out = _kernel_call(x.reshape(_B * _F * _P))
    return out.reshape(_B, _F, _K)


# speedup vs baseline: 1.2317x; 1.0682x over previous
import jax
import jax.numpy as jnp
from jax import lax
from jax.experimental import pallas as pl
from jax.experimental.pallas import tpu as pltpu
from jax.experimental.pallas import tpu_sc as plsc

_B, _F, _P = 128, 4, 32768
_K = 1024
_NW = 32
_RPW = _B // _NW
_NVROW = _P // 16
_U = 8

_mesh = plsc.VectorSubcoreMesh(core_axis_name="c", subcore_axis_name="s")


def _body(xf_hbm, out_hbm, v_ref, ca_ref, cb_ref, hist_ref, base_ref,
          ka_ref, ia_ref, kb_ref, ib_ref, gi_ref, gd_ref, sem):
    wid = lax.axis_index("s") * 2 + lax.axis_index("c")
    iota = lax.iota(jnp.int32, 16)
    ones = jnp.ones(16, jnp.int32)
    zeros = jnp.zeros(16, jnp.int32)

    def keys_of(v):
        b = plsc.bitcast(v, jnp.int32)
        flip = (b >> 31) | jnp.int32(-2147483648)
        return plsc.bitcast(b ^ flip, jnp.uint32)

    def zero_hist():
        for j in range(16):
            hist_ref[pl.ds(j * 16, 16)] = zeros

    def find_bin(p):
        z = jnp.int32(0)
        found, bbin, below, cnt_at, run = z, z, z, z, z
        for j in range(16):
            h = hist_ref[pl.ds(j * 16, 16)]
            cum = plsc.cumsum(h) + run
            ffs = jnp.max(plsc.all_reduce_ffs(cum > p))
            newly = (found == 0) & (ffs < 16)
            lane_cnt = jnp.sum(jnp.where(iota == ffs, h, 0))
            lane_cum = jnp.sum(jnp.where(iota == ffs, cum, 0))
            bbin = jnp.where(newly, j * 16 + ffs, bbin)
            below = jnp.where(newly, lane_cum - lane_cnt, below)
            cnt_at = jnp.where(newly, lane_cnt, cnt_at)
            run = run + jnp.sum(h)
            found = found | newly.astype(jnp.int32)
        return bbin, below, cnt_at

    def per_row(rr, row_carry):
        row = wid * _RPW + rr
        row_base = row * (_F * _P)
        pltpu.sync_copy(xf_hbm.at[pl.ds(row_base, _P)], v_ref)

        zero_hist()

        def h0(j, c):
            for u in range(_U):
                k = keys_of(v_ref[pl.ds((j * _U + u) * 16, 16)])
                plsc.addupdate_scatter(
                    hist_ref, [(k >> 24).astype(jnp.int32)], ones)
            return c

        lax.fori_loop(0, _NVROW // _U, h0, jnp.int32(0))

        above = jnp.int32(0)
        cn = jnp.int32(_P)
        bbin, below, cnt_at = find_bin(cn - (_K - above))
        above = above + (cn - below - cnt_at)
        cn = cnt_at
        t_key = bbin.astype(jnp.uint32) << 24

        def c0(j, w):
            for u in range(_U):
                k = keys_of(v_ref[pl.ds((j * _U + u) * 16, 16)])
                m = (k >> 24).astype(jnp.int32) == bbin
                cs = plsc.cumsum(m.astype(jnp.int32))
                plsc.store_scatter(ca_ref, [w + cs - 1],
                                   plsc.bitcast(k, jnp.int32), mask=m)
                w = w + jnp.max(cs)
            return w

        lax.fori_loop(0, _NVROW // _U, c0, jnp.int32(0))

        for lvl, shift in enumerate((16, 8, 0)):
            src = (ca_ref, cb_ref)[lvl % 2]
            dst = (cb_ref, ca_ref)[lvl % 2]
            nv = (cn + 16 * _U - 1) // (16 * _U)
            zero_hist()

            def hl(j, c, src=src, shift=shift, cn=cn):
                for u in range(_U):
                    base = (j * _U + u) * 16
                    k = plsc.bitcast(src[pl.ds(base, 16)], jnp.uint32)
                    valid = (base + iota) < cn
                    b = ((k >> shift) & 0xFF).astype(jnp.int32)
                    plsc.addupdate_scatter(hist_ref, [b], ones, mask=valid)
                return c

            lax.fori_loop(0, nv, hl, jnp.int32(0))
            bbin, below, cnt_at = find_bin(cn - (_K - above))

            if shift != 0:
                def cl(j, w, src=src, dst=dst, shift=shift, cn=cn,
                       bbin=bbin):
                    for u in range(_U):
                        base = (j * _U + u) * 16
                        ki = src[pl.ds(base, 16)]
                        k = plsc.bitcast(ki, jnp.uint32)
                        valid = (base + iota) < cn
                        m = valid & (((k >> shift) & 0xFF).astype(jnp.int32)
                                     == bbin)
                        cs = plsc.cumsum(m.astype(jnp.int32))
                        plsc.store_scatter(dst, [w + cs - 1], ki, mask=m)
                        w = w + jnp.max(cs)
                    return w

                lax.fori_loop(0, nv, cl, jnp.int32(0))

            above = above + (cn - below - cnt_at)
            cn = cnt_at
            t_key = t_key | (bbin.astype(jnp.uint32) << shift)

        t_skip = cn - (_K - above)

        def sel1(j, w):
            for u in range(_U):
                base = (j * _U + u) * 16
                k = keys_of(v_ref[pl.ds(base, 16)])
                m = k >= t_key
                cs = plsc.cumsum(m.astype(jnp.int32))
                pos = w + cs - 1
                plsc.store_scatter(ca_ref, [pos],
                                   plsc.bitcast(k, jnp.int32), mask=m)
                plsc.store_scatter(cb_ref, [pos], base + iota, mask=m)
                w = w + jnp.max(cs)
            return w

        cnc = lax.fori_loop(0, _NVROW // _U, sel1, jnp.int32(0))

        def sel2(j, c):
            ts, ns = c
            for u in range(_U):
                base = (j * _U + u) * 16
                k = plsc.bitcast(ca_ref[pl.ds(base, 16)], jnp.uint32)
                ix = cb_ref[pl.ds(base, 16)]
                valid = (base + iota) < cnc
                m_eq = valid & (k == t_key)
                ce = plsc.cumsum(m_eq.astype(jnp.int32))
                keep = (valid & (k > t_key)) | (
                    m_eq & ((ts + ce - 1) >= t_skip))
                ck = plsc.cumsum(keep.astype(jnp.int32))
                pos = ns + ck - 1
                plsc.store_scatter(ka_ref, [pos],
                                   plsc.bitcast(k, jnp.int32), mask=keep)
                plsc.store_scatter(ia_ref, [pos], ix, mask=keep)
                ts = ts + jnp.max(ce)
                ns = ns + jnp.max(ck)
            return ts, ns

        lax.fori_loop(0, (cnc + 16 * _U - 1) // (16 * _U), sel2,
                      (jnp.int32(0), jnp.int32(0)))

        bufs = ((ka_ref, ia_ref), (kb_ref, ib_ref))
        for p in range(4):
            src_k, src_i = bufs[p % 2]
            dst_k, dst_i = bufs[(p + 1) % 2]
            shift = 8 * p
            zero_hist()

            def hp(j, c, src_k=src_k, shift=shift):
                for u in range(_U):
                    k = plsc.bitcast(
                        src_k[pl.ds((j * _U + u) * 16, 16)], jnp.uint32)
                    b = ((k >> shift) & 0xFF).astype(jnp.int32)
                    plsc.addupdate_scatter(hist_ref, [b], ones)
                return c

            lax.fori_loop(0, _K // 16 // _U, hp, jnp.int32(0))

            run = jnp.int32(0)
            for j in range(16):
                h = hist_ref[pl.ds(j * 16, 16)]
                cum = plsc.cumsum(h) + run
                base_ref[pl.ds(j * 16, 16)] = cum - h
                run = jnp.max(cum)

            def sc(j, c, src_k=src_k, src_i=src_i, dst_k=dst_k,
                   dst_i=dst_i, shift=shift):
                for u in range(_U):
                    base = (j * _U + u) * 16
                    ki = src_k[pl.ds(base, 16)]
                    ix = src_i[pl.ds(base, 16)]
                    b = ((plsc.bitcast(ki, jnp.uint32) >> shift)
                         & 0xFF).astype(jnp.int32)
                    cnt, _ = plsc.scan_count(b)
                    pos = plsc.load_gather(base_ref, [b]) + cnt - 1
                    plsc.store_scatter(dst_k, [pos], ki)
                    plsc.store_scatter(dst_i, [pos], ix)
                    plsc.addupdate_scatter(base_ref, [b], ones)
                return c

            lax.fori_loop(0, _K // 16 // _U, sc, jnp.int32(0))

        def gx(j, c):
            for u in range(_U):
                base = (j * _U + u) * 16
                ix = ia_ref[pl.ds(base, 16)]
                for f in range(_F):
                    gi_ref[pl.ds(f * _K + base, 16)] = (
                        ix + (row_base + f * _P))
            return c

        lax.fori_loop(0, _K // 16 // _U, gx, jnp.int32(0))
        copies = [
            pltpu.async_copy(
                xf_hbm.at[gi_ref.at[pl.ds(c * 128, 128)]],
                gd_ref.at[pl.ds(c * 128, 128)], sem)
            for c in range(_F * _K // 128)
        ]
        for cp in copies:
            cp.wait()
        pltpu.sync_copy(gd_ref,
                        out_hbm.at[pl.ds(row * _F * _K, _F * _K)])
        return row_carry

    lax.fori_loop(0, _RPW, per_row, jnp.int32(0))


_kernel_call = pl.kernel(
    _body,
    out_type=jax.ShapeDtypeStruct((_B * _F * _K,), jnp.float32),
    mesh=_mesh,
    compiler_params=pltpu.CompilerParams(needs_layout_passes=False),
    scratch_types=[
        pltpu.VMEM((_P,), jnp.float32),
        pltpu.VMEM((_P,), jnp.int32),
        pltpu.VMEM((_P,), jnp.int32),
        pltpu.VMEM((256,), jnp.int32),
        pltpu.VMEM((256,), jnp.int32),
        pltpu.VMEM((_K,), jnp.int32),
        pltpu.VMEM((_K,), jnp.int32),
        pltpu.VMEM((_K,), jnp.int32),
        pltpu.VMEM((_K,), jnp.int32),
        pltpu.VMEM((_F * _K,), jnp.int32),
        pltpu.VMEM((_F * _K,), jnp.float32),
        pltpu.SemaphoreType.DMA,
    ],
)


@jax.jit
def kernel(x):
    out = _kernel_call(x.reshape(_B * _F * _P))
    return out.reshape(_B, _F, _K)
